# Initial kernel scaffold; baseline (speedup 1.0000x reference)
import jax
import jax.numpy as jnp
from jax.experimental import pallas as pl


def kernel(user_embeddings, item_embeddings, edge_index, edge_values):
    raise NotImplementedError("write your pallas kernel here")


# speedup vs baseline: 1.8433x; 1.8433x over previous
import functools

import jax
import jax.numpy as jnp
from jax import lax
from jax.experimental import pallas as pl
from jax.experimental.pallas import tpu as pltpu
from jax.experimental.pallas import tpu_sc as plsc

_N = 50000
_D = 64
_DH = 32
_E = 800000
_NS = 16
_EPT = _E // _NS
_C = 80
_NCHUNK = _EPT // _C
_NP = 50048
_RPT = _NP // _NS
_ZC = 184
_NZ = _RPT // _ZC

_f32 = jnp.float32
_i32 = jnp.int32


def _body(row_hbm, col_hbm, val_hbm, u0, i0, out_u, out_i,
          t1, t2, s1, s2,
          acc, colv, dstv, valv, rows, zbuf, wbuf, b1, b2):
    h = lax.axis_index("c")
    tid = lax.axis_index("s")
    hoff = jnp.full((16,), h * _NP, _i32)

    def zinit(r, c):
        z16 = jnp.zeros((16,), _f32)
        zbuf[r, pl.ds(0, 16)] = z16
        zbuf[r, pl.ds(16, 16)] = z16
        return c
    lax.fori_loop(0, _ZC, zinit, 0)

    def spmm(src, dst, fold=None):
        def zero_chunk(z, c):
            pltpu.sync_copy(zbuf, acc.at[pl.ds(tid * _RPT + z * _ZC, _ZC)])
            return c
        lax.fori_loop(0, _NZ, zero_chunk, 0)
        plsc.subcore_barrier()

        def edge_chunk(k, c):
            base = tid * _EPT + k * _C
            pltpu.sync_copy(col_hbm.at[pl.ds(base, _C)], colv)
            pltpu.sync_copy(row_hbm.at[pl.ds(base, _C)], dstv)
            pltpu.sync_copy(val_hbm.at[pl.ds(base, _C)], valv)
            for g in range(_C // 16):
                colv[pl.ds(g * 16, 16)] = colv[pl.ds(g * 16, 16)] + hoff
            pltpu.sync_copy(src.at[colv], rows)
            def scale_group(g, c):
                vv = valv[pl.ds(g * 16, 16)]
                lane = jnp.zeros((16,), _i32)
                one = jnp.full((16,), 1, _i32)
                for l in range(16):
                    e = g * 16 + l
                    sp = lax.gather(
                        vv, lane.reshape(16, 1),
                        lax.GatherDimensionNumbers(
                            offset_dims=(), collapsed_slice_dims=(0,),
                            start_index_map=(0,)),
                        (1,), mode=lax.GatherScatterMode.PROMISE_IN_BOUNDS)
                    rows[e, pl.ds(0, 16)] = rows[e, pl.ds(0, 16)] * sp
                    rows[e, pl.ds(16, 16)] = rows[e, pl.ds(16, 16)] * sp
                    lane = lane + one
                return c
            lax.fori_loop(0, _C // 16, scale_group, 0)
            pltpu.sync_copy(rows, acc.at[dstv], add=True)
            return c
        lax.fori_loop(0, _NCHUNK, edge_chunk, 0)
        plsc.subcore_barrier()

        def wb_chunk(z, c):
            r0 = tid * _RPT + z * _ZC
            pltpu.sync_copy(acc.at[pl.ds(r0, _ZC)], wbuf)
            if fold is not None:
                fa, fb = fold
                pltpu.sync_copy(fa.at[pl.ds(h * _NP + r0, _ZC)], b1)
                pltpu.sync_copy(fb.at[pl.ds(h * _NP + r0, _ZC)], b2)
                def fold_row(r, c):
                    third = jnp.full((16,), 1.0 / 3.0, _f32)
                    for c0 in (0, 16):
                        s = (wbuf[r, pl.ds(c0, 16)] + b1[r, pl.ds(c0, 16)]
                             + b2[r, pl.ds(c0, 16)])
                        wbuf[r, pl.ds(c0, 16)] = s * third
                    return c
                lax.fori_loop(0, _ZC, fold_row, 0)
            pltpu.sync_copy(wbuf, dst.at[pl.ds(h * _NP + r0, _ZC)])
            return c
        lax.fori_loop(0, _NZ, wb_chunk, 0)
        plsc.subcore_barrier()

    spmm(i0, t1)
    spmm(u0, s1)
    spmm(s1, t2)
    spmm(t1, s2)
    spmm(s2, out_u, fold=(t1, t2))
    spmm(t2, out_i, fold=(s1, s2))


_sds = jax.ShapeDtypeStruct

_gcn = functools.partial(
    pl.kernel,
    out_type=(_sds((2 * _NP, _DH), _f32), _sds((2 * _NP, _DH), _f32)),
    mesh=plsc.VectorSubcoreMesh(core_axis_name="c", subcore_axis_name="s"),
    compiler_params=pltpu.CompilerParams(use_tc_tiling_on_sc=False),
    scratch_types=[
        pltpu.HBM((2 * _NP, _DH), _f32),
        pltpu.HBM((2 * _NP, _DH), _f32),
        pltpu.HBM((2 * _NP, _DH), _f32),
        pltpu.HBM((2 * _NP, _DH), _f32),
        pltpu.VMEM_SHARED((_NP, _DH), _f32),
        pltpu.VMEM((_C,), _i32),
        pltpu.VMEM((_C,), _i32),
        pltpu.VMEM((_C,), _f32),
        pltpu.VMEM((_C, _DH), _f32),
        pltpu.VMEM((_ZC, _DH), _f32),
        pltpu.VMEM((_ZC, _DH), _f32),
        pltpu.VMEM((_ZC, _DH), _f32),
        pltpu.VMEM((_ZC, _DH), _f32),
    ],
)(_body)


def kernel(user_embeddings, item_embeddings, edge_index, edge_values):
    row = edge_index[0].astype(_i32)
    col = edge_index[1].astype(_i32)
    def stack(t):
        s = jnp.zeros((2 * _NP, _DH), _f32)
        return s.at[:_N].set(t[:, :_DH]).at[_NP:_NP + _N].set(t[:, _DH:])
    out_u, out_i = _gcn(row, col, edge_values,
                        stack(user_embeddings), stack(item_embeddings))
    new_user = jnp.concatenate([out_u[:_N], out_u[_NP:_NP + _N]], axis=1)
    new_item = jnp.concatenate([out_i[:_N], out_i[_NP:_NP + _N]], axis=1)
    return (new_user, new_item)


# speedup vs baseline: 5.7504x; 3.1196x over previous
import functools

import jax
import jax.numpy as jnp
from jax import lax
from jax.experimental import pallas as pl
from jax.experimental.pallas import tpu as pltpu
from jax.experimental.pallas import tpu_sc as plsc

_N = 50000
_D = 64
_DH = 32
_E = 800000
_NS = 16
_EPT = _E // _NS
_C = 80
_CH = 25
_NSUB = _EPT // (_CH * _C)
_NP = 50048
_RPT = _NP // _NS
_ZC = 136
_NZ = _RPT // _ZC

_f32 = jnp.float32
_i32 = jnp.int32


def _body(row_hbm, col_hbm, val_hbm, u0, i0, out_u, out_i,
          t1, t2, s1, s2,
          acc, col2d, dst2d, val2d, rows0, rows1, zw, b1, b2,
          sem_g0, sem_g1):
    h = lax.axis_index("c")
    tid = lax.axis_index("s")
    hoff = jnp.full((16,), h * _NP, _i32)

    def spmm(src, dst, fold=None):
        def zinit(r, c):
            z16 = jnp.zeros((16,), _f32)
            zw[r, pl.ds(0, 16)] = z16
            zw[r, pl.ds(16, 16)] = z16
            return c
        lax.fori_loop(0, _ZC, zinit, 0)

        def zero_chunk(z, c):
            pltpu.sync_copy(zw, acc.at[pl.ds(tid * _RPT + z * _ZC, _ZC)])
            return c
        lax.fori_loop(0, _NZ, zero_chunk, 0)
        plsc.subcore_barrier()

        def scale(rows, j):
            def scale_group(g, c):
                vv = val2d[j, pl.ds(g * 16, 16)]
                lane = jnp.zeros((16,), _i32)
                one = jnp.full((16,), 1, _i32)
                for l in range(16):
                    e = g * 16 + l
                    sp = lax.gather(
                        vv, lane.reshape(16, 1),
                        lax.GatherDimensionNumbers(
                            offset_dims=(), collapsed_slice_dims=(0,),
                            start_index_map=(0,)),
                        (1,), mode=lax.GatherScatterMode.PROMISE_IN_BOUNDS)
                    rows[e, pl.ds(0, 16)] = rows[e, pl.ds(0, 16)] * sp
                    rows[e, pl.ds(16, 16)] = rows[e, pl.ds(16, 16)] * sp
                    lane = lane + one
                return c
            lax.fori_loop(0, _C // 16, scale_group, 0)

        def comp(rows, sem, j):
            pltpu.make_async_copy(src.at[col2d.at[j]], rows, sem).wait()
            scale(rows, j)
            pltpu.sync_copy(rows, acc.at[dst2d.at[j]], add=True)

        def subpass(s, c):
            r0 = tid * (_CH * _NSUB) + s * _CH
            pltpu.sync_copy(col_hbm.at[pl.ds(r0, _CH)], col2d)
            pltpu.sync_copy(row_hbm.at[pl.ds(r0, _CH)], dst2d)
            pltpu.sync_copy(val_hbm.at[pl.ds(r0, _CH)], val2d)

            def adjust(r, cc):
                for g in range(_C // 16):
                    col2d[r, pl.ds(g * 16, 16)] = (
                        col2d[r, pl.ds(g * 16, 16)] + hoff)
                return cc
            lax.fori_loop(0, _CH, adjust, 0)

            pltpu.async_copy(src.at[col2d.at[0]], rows0, sem_g0)

            def pair(jj, cc):
                j0 = 2 * jj
                pltpu.async_copy(src.at[col2d.at[j0 + 1]], rows1, sem_g1)
                comp(rows0, sem_g0, j0)
                pltpu.async_copy(src.at[col2d.at[j0 + 2]], rows0, sem_g0)
                comp(rows1, sem_g1, j0 + 1)
                return cc
            lax.fori_loop(0, (_CH - 1) // 2, pair, 0)
            comp(rows0, sem_g0, _CH - 1)
            return c
        lax.fori_loop(0, _NSUB, subpass, 0)
        plsc.subcore_barrier()

        def wb_chunk(z, c):
            r0 = tid * _RPT + z * _ZC
            pltpu.sync_copy(acc.at[pl.ds(r0, _ZC)], zw)
            if fold is not None:
                fa, fb = fold
                pltpu.sync_copy(fa.at[pl.ds(h * _NP + r0, _ZC)], b1)
                pltpu.sync_copy(fb.at[pl.ds(h * _NP + r0, _ZC)], b2)
                def fold_row(r, c):
                    third = jnp.full((16,), 1.0 / 3.0, _f32)
                    for c0 in (0, 16):
                        s = (zw[r, pl.ds(c0, 16)] + b1[r, pl.ds(c0, 16)]
                             + b2[r, pl.ds(c0, 16)])
                        zw[r, pl.ds(c0, 16)] = s * third
                    return c
                lax.fori_loop(0, _ZC, fold_row, 0)
            pltpu.sync_copy(zw, dst.at[pl.ds(h * _NP + r0, _ZC)])
            return c
        lax.fori_loop(0, _NZ, wb_chunk, 0)
        plsc.subcore_barrier()

    spmm(i0, t1)
    spmm(u0, s1)
    spmm(s1, t2)
    spmm(t1, s2)
    spmm(s2, out_u, fold=(t1, t2))
    spmm(t2, out_i, fold=(s1, s2))


_sds = jax.ShapeDtypeStruct

_gcn = functools.partial(
    pl.kernel,
    out_type=(_sds((2 * _NP, _DH), _f32), _sds((2 * _NP, _DH), _f32)),
    mesh=plsc.VectorSubcoreMesh(core_axis_name="c", subcore_axis_name="s"),
    compiler_params=pltpu.CompilerParams(use_tc_tiling_on_sc=False),
    scratch_types=[
        pltpu.HBM((2 * _NP, _DH), _f32),
        pltpu.HBM((2 * _NP, _DH), _f32),
        pltpu.HBM((2 * _NP, _DH), _f32),
        pltpu.HBM((2 * _NP, _DH), _f32),
        pltpu.VMEM_SHARED((_NP, _DH), _f32),
        pltpu.VMEM((_CH, _C), _i32),
        pltpu.VMEM((_CH, _C), _i32),
        pltpu.VMEM((_CH, _C), _f32),
        pltpu.VMEM((_C, _DH), _f32),
        pltpu.VMEM((_C, _DH), _f32),
        pltpu.VMEM((_ZC, _DH), _f32),
        pltpu.VMEM((_ZC, _DH), _f32),
        pltpu.VMEM((_ZC, _DH), _f32),
        pltpu.SemaphoreType.DMA,
        pltpu.SemaphoreType.DMA,
    ],
)(_body)


def kernel(user_embeddings, item_embeddings, edge_index, edge_values):
    row = edge_index[0].astype(_i32).reshape(_E // _C, _C)
    col = edge_index[1].astype(_i32).reshape(_E // _C, _C)
    val2 = edge_values.reshape(_E // _C, _C)
    def stack(t):
        s = jnp.zeros((2 * _NP, _DH), _f32)
        return s.at[:_N].set(t[:, :_DH]).at[_NP:_NP + _N].set(t[:, _DH:])
    out_u, out_i = _gcn(row, col, val2,
                        stack(user_embeddings), stack(item_embeddings))
    new_user = jnp.concatenate([out_u[:_N], out_u[_NP:_NP + _N]], axis=1)
    new_item = jnp.concatenate([out_i[:_N], out_i[_NP:_NP + _N]], axis=1)
    return (new_user, new_item)


# speedup vs baseline: 7.6754x; 1.3348x over previous
import functools

import jax
import jax.numpy as jnp
from jax import lax
from jax.experimental import pallas as pl
from jax.experimental.pallas import tpu as pltpu
from jax.experimental.pallas import tpu_sc as plsc

_N = 50000
_D = 64
_DH = 32
_E = 800000
_NS = 16
_EPT = _E // _NS
_C = 80
_CH = 25
_NSUB = _EPT // (_CH * _C)
_NP = 50048
_RPT = _NP // _NS
_ZC = 136
_NZ = _RPT // _ZC

_f32 = jnp.float32
_i32 = jnp.int32


def _body(row_hbm, col_hbm, val_hbm, u0, i0, out_u, out_i,
          t1, t2, s1, s2,
          acc, col2d, dst2d, val2d, rows0, rows1, rows2, rows3, zw, b1, b2,
          sem_g0, sem_g1, sem_g2, sem_g3, sem_s0, sem_s1, sem_s2, sem_s3):
    h = lax.axis_index("c")
    tid = lax.axis_index("s")
    hoff = jnp.full((16,), h * _NP, _i32)

    def spmm(src, dst, fold=None):
        def zinit(r, c):
            z16 = jnp.zeros((16,), _f32)
            zw[r, pl.ds(0, 16)] = z16
            zw[r, pl.ds(16, 16)] = z16
            return c
        lax.fori_loop(0, _ZC, zinit, 0)

        def zero_chunk(z, c):
            pltpu.sync_copy(zw, acc.at[pl.ds(tid * _RPT + z * _ZC, _ZC)])
            return c
        lax.fori_loop(0, _NZ, zero_chunk, 0)
        plsc.subcore_barrier()

        def scale(rows, j):
            def scale_group(g, c):
                vv = val2d[j, pl.ds(g * 16, 16)]
                lane = jnp.zeros((16,), _i32)
                one = jnp.full((16,), 1, _i32)
                for l in range(16):
                    e = g * 16 + l
                    sp = lax.gather(
                        vv, lane.reshape(16, 1),
                        lax.GatherDimensionNumbers(
                            offset_dims=(), collapsed_slice_dims=(0,),
                            start_index_map=(0,)),
                        (1,), mode=lax.GatherScatterMode.PROMISE_IN_BOUNDS)
                    rows[e, pl.ds(0, 16)] = rows[e, pl.ds(0, 16)] * sp
                    rows[e, pl.ds(16, 16)] = rows[e, pl.ds(16, 16)] * sp
                    lane = lane + one
                return c
            lax.fori_loop(0, _C // 16, scale_group, 0)

        bufs = ((rows0, sem_g0, sem_s0), (rows1, sem_g1, sem_s1),
                (rows2, sem_g2, sem_s2), (rows3, sem_g3, sem_s3))

        def wait_scat(b, j):
            rows, _, sem_s = bufs[b]
            pltpu.make_async_copy(rows, acc.at[dst2d.at[j]], sem_s).wait()

        def comp(b, j):
            rows, sem_g, sem_s = bufs[b]
            pltpu.make_async_copy(src.at[col2d.at[j]], rows, sem_g).wait()
            scale(rows, j)
            pltpu.async_copy(rows, acc.at[dst2d.at[j]], sem_s, add=True)

        def gath(b, j):
            rows, sem_g, _ = bufs[b]
            pltpu.async_copy(src.at[col2d.at[j]], rows, sem_g)

        def subpass(s, c):
            r0 = tid * (_CH * _NSUB) + s * _CH
            pltpu.sync_copy(col_hbm.at[pl.ds(r0, _CH)], col2d)
            pltpu.sync_copy(row_hbm.at[pl.ds(r0, _CH)], dst2d)
            pltpu.sync_copy(val_hbm.at[pl.ds(r0, _CH)], val2d)

            def adjust(r, cc):
                for g in range(_C // 16):
                    col2d[r, pl.ds(g * 16, 16)] = (
                        col2d[r, pl.ds(g * 16, 16)] + hoff)
                return cc
            lax.fori_loop(0, _CH, adjust, 0)

            gath(0, 0)
            gath(1, 1)

            def quad(q, cc):
                for b in range(4):
                    j = 4 * q + b
                    if b >= 2:
                        wait_scat((b + 2) % 4, j - 2)
                    else:
                        @pl.when(q > 0)
                        def _():
                            wait_scat((b + 2) % 4, j - 2)
                    @pl.when(j + 2 < _CH)
                    def _():
                        gath((b + 2) % 4, j + 2)
                    comp(b, j)
                return cc
            lax.fori_loop(0, _CH // 4, quad, 0)

            wait_scat(2, _CH - 3)
            comp(0, _CH - 1)
            wait_scat(3, _CH - 2)
            wait_scat(0, _CH - 1)
            return c
        lax.fori_loop(0, _NSUB, subpass, 0)
        plsc.subcore_barrier()

        def wb_chunk(z, c):
            r0 = tid * _RPT + z * _ZC
            pltpu.sync_copy(acc.at[pl.ds(r0, _ZC)], zw)
            if fold is not None:
                fa, fb = fold
                pltpu.sync_copy(fa.at[pl.ds(h * _NP + r0, _ZC)], b1)
                pltpu.sync_copy(fb.at[pl.ds(h * _NP + r0, _ZC)], b2)
                def fold_row(r, c):
                    third = jnp.full((16,), 1.0 / 3.0, _f32)
                    for c0 in (0, 16):
                        s = (zw[r, pl.ds(c0, 16)] + b1[r, pl.ds(c0, 16)]
                             + b2[r, pl.ds(c0, 16)])
                        zw[r, pl.ds(c0, 16)] = s * third
                    return c
                lax.fori_loop(0, _ZC, fold_row, 0)
            pltpu.sync_copy(zw, dst.at[pl.ds(h * _NP + r0, _ZC)])
            return c
        lax.fori_loop(0, _NZ, wb_chunk, 0)
        plsc.subcore_barrier()

    spmm(i0, t1)
    spmm(u0, s1)
    spmm(s1, t2)
    spmm(t1, s2)
    spmm(s2, out_u, fold=(t1, t2))
    spmm(t2, out_i, fold=(s1, s2))


_sds = jax.ShapeDtypeStruct

_gcn = functools.partial(
    pl.kernel,
    out_type=(_sds((2 * _NP, _DH), _f32), _sds((2 * _NP, _DH), _f32)),
    mesh=plsc.VectorSubcoreMesh(core_axis_name="c", subcore_axis_name="s"),
    compiler_params=pltpu.CompilerParams(use_tc_tiling_on_sc=False),
    scratch_types=[
        pltpu.HBM((2 * _NP, _DH), _f32),
        pltpu.HBM((2 * _NP, _DH), _f32),
        pltpu.HBM((2 * _NP, _DH), _f32),
        pltpu.HBM((2 * _NP, _DH), _f32),
        pltpu.VMEM_SHARED((_NP, _DH), _f32),
        pltpu.VMEM((_CH, _C), _i32),
        pltpu.VMEM((_CH, _C), _i32),
        pltpu.VMEM((_CH, _C), _f32),
        pltpu.VMEM((_C, _DH), _f32),
        pltpu.VMEM((_C, _DH), _f32),
        pltpu.VMEM((_C, _DH), _f32),
        pltpu.VMEM((_C, _DH), _f32),
        pltpu.VMEM((_ZC, _DH), _f32),
        pltpu.VMEM((_ZC, _DH), _f32),
        pltpu.VMEM((_ZC, _DH), _f32),
        pltpu.SemaphoreType.DMA,
        pltpu.SemaphoreType.DMA,
        pltpu.SemaphoreType.DMA,
        pltpu.SemaphoreType.DMA,
        pltpu.SemaphoreType.DMA,
        pltpu.SemaphoreType.DMA,
        pltpu.SemaphoreType.DMA,
        pltpu.SemaphoreType.DMA,
    ],
)(_body)


def kernel(user_embeddings, item_embeddings, edge_index, edge_values):
    row = edge_index[0].astype(_i32).reshape(_E // _C, _C)
    col = edge_index[1].astype(_i32).reshape(_E // _C, _C)
    val2 = edge_values.reshape(_E // _C, _C)
    def stack(t):
        s = jnp.zeros((2 * _NP, _DH), _f32)
        return s.at[:_N].set(t[:, :_DH]).at[_NP:_NP + _N].set(t[:, _DH:])
    out_u, out_i = _gcn(row, col, val2,
                        stack(user_embeddings), stack(item_embeddings))
    new_user = jnp.concatenate([out_u[:_N], out_u[_NP:_NP + _N]], axis=1)
    new_item = jnp.concatenate([out_i[:_N], out_i[_NP:_NP + _N]], axis=1)
    return (new_user, new_item)


# speedup vs baseline: 9.7547x; 1.2709x over previous
import functools

import jax
import jax.numpy as jnp
from jax import lax
from jax.experimental import pallas as pl
from jax.experimental.pallas import tpu as pltpu
from jax.experimental.pallas import tpu_sc as plsc

_N = 50000
_D = 64
_DH = 32
_E = 800000
_NS = 16
_EPT = _E // _NS
_C = 80
_CH = 125
_NSUB = _EPT // (_CH * _C)
_NP = 50048
_RPT = _NP // _NS
_ZC = 136
_NZ = _RPT // _ZC

_f32 = jnp.float32
_bf16 = jnp.bfloat16
_i32 = jnp.int32


def _body(row_hbm, col_hbm, val_hbm, u0, i0, out_u, out_i,
          t1, t2, s1, s2,
          acc, col2d, dst2d, val2d, rows0, rows1, rows2, rows3, zw, b1, b2,
          sem_g0, sem_g1, sem_g2, sem_g3, sem_s0, sem_s1, sem_s2, sem_s3):
    h = lax.axis_index("c")
    tid = lax.axis_index("s")
    hoff = jnp.full((16,), h * _NP, _i32)

    def spmm(src, dst, fold=None):
        def zinit(r, c):
            zw[r, :] = jnp.zeros((_DH,), _bf16)
            return c
        lax.fori_loop(0, _ZC, zinit, 0)

        def zero_chunk(z, c):
            pltpu.sync_copy(zw, acc.at[pl.ds(tid * _RPT + z * _ZC, _ZC)])
            return c
        lax.fori_loop(0, _NZ, zero_chunk, 0)
        plsc.subcore_barrier()

        def scale(rows, j):
            def scale_group(g, c):
                vv = val2d[j, pl.ds(g * 16, 16)]
                yb = plsc.bitcast(vv, _i32)
                yr = yb + jnp.full((16,), 0x8000, _i32)
                t = lax.bitwise_and(yr, jnp.full((16,), -0x10000, _i32))
                pairs = lax.bitwise_or(
                    t, lax.shift_right_logical(t, jnp.full((16,), 16, _i32)))
                lane = jnp.zeros((16,), _i32)
                one = jnp.full((16,), 1, _i32)
                for l in range(16):
                    e = g * 16 + l
                    spi = lax.gather(
                        pairs, lane.reshape(16, 1),
                        lax.GatherDimensionNumbers(
                            offset_dims=(), collapsed_slice_dims=(0,),
                            start_index_map=(0,)),
                        (1,), mode=lax.GatherScatterMode.PROMISE_IN_BOUNDS)
                    spb = plsc.bitcast(spi, _bf16)
                    rows[e, :] = rows[e, :] * spb
                    lane = lane + one
                return c
            lax.fori_loop(0, _C // 16, scale_group, 0)

        bufs = ((rows0, sem_g0, sem_s0), (rows1, sem_g1, sem_s1),
                (rows2, sem_g2, sem_s2), (rows3, sem_g3, sem_s3))

        def wait_scat(b, j):
            rows, _, sem_s = bufs[b]
            pltpu.make_async_copy(rows, acc.at[dst2d.at[j]], sem_s).wait()

        def comp(b, j):
            rows, sem_g, sem_s = bufs[b]
            pltpu.make_async_copy(src.at[col2d.at[j]], rows, sem_g).wait()
            scale(rows, j)
            pltpu.async_copy(rows, acc.at[dst2d.at[j]], sem_s, add=True)

        def gath(b, j):
            rows, sem_g, _ = bufs[b]
            pltpu.async_copy(src.at[col2d.at[j]], rows, sem_g)

        def subpass(s, c):
            r0 = tid * (_CH * _NSUB) + s * _CH
            pltpu.sync_copy(col_hbm.at[pl.ds(r0, _CH)], col2d)
            pltpu.sync_copy(row_hbm.at[pl.ds(r0, _CH)], dst2d)
            pltpu.sync_copy(val_hbm.at[pl.ds(r0, _CH)], val2d)

            def adjust(r, cc):
                for g in range(_C // 16):
                    col2d[r, pl.ds(g * 16, 16)] = (
                        col2d[r, pl.ds(g * 16, 16)] + hoff)
                return cc
            lax.fori_loop(0, _CH, adjust, 0)

            gath(0, 0)
            gath(1, 1)

            def quad(q, cc):
                for b in range(4):
                    j = 4 * q + b
                    if b >= 2:
                        wait_scat((b + 2) % 4, j - 2)
                    else:
                        @pl.when(q > 0)
                        def _():
                            wait_scat((b + 2) % 4, j - 2)
                    @pl.when(j + 2 < _CH)
                    def _():
                        gath((b + 2) % 4, j + 2)
                    comp(b, j)
                return cc
            lax.fori_loop(0, _CH // 4, quad, 0)

            wait_scat(2, _CH - 3)
            comp(0, _CH - 1)
            wait_scat(3, _CH - 2)
            wait_scat(0, _CH - 1)
            return c
        lax.fori_loop(0, _NSUB, subpass, 0)
        plsc.subcore_barrier()

        def wb_chunk(z, c):
            r0 = tid * _RPT + z * _ZC
            pltpu.sync_copy(acc.at[pl.ds(r0, _ZC)], zw)
            if fold is not None:
                fa, fb = fold
                pltpu.sync_copy(fa.at[pl.ds(h * _NP + r0, _ZC)], b1)
                pltpu.sync_copy(fb.at[pl.ds(h * _NP + r0, _ZC)], b2)
                def fold_row(r, c):
                    third = jnp.full((_DH,), 1.0 / 3.0, _bf16)
                    zw[r, :] = (zw[r, :] + b1[r, :] + b2[r, :]) * third
                    return c
                lax.fori_loop(0, _ZC, fold_row, 0)
            pltpu.sync_copy(zw, dst.at[pl.ds(h * _NP + r0, _ZC)])
            return c
        lax.fori_loop(0, _NZ, wb_chunk, 0)
        plsc.subcore_barrier()

    spmm(i0, t1)
    spmm(u0, s1)
    spmm(s1, t2)
    spmm(t1, s2)
    spmm(s2, out_u, fold=(t1, t2))
    spmm(t2, out_i, fold=(s1, s2))


_sds = jax.ShapeDtypeStruct

_gcn = functools.partial(
    pl.kernel,
    out_type=(_sds((2 * _NP, _DH), _bf16), _sds((2 * _NP, _DH), _bf16)),
    mesh=plsc.VectorSubcoreMesh(core_axis_name="c", subcore_axis_name="s"),
    compiler_params=pltpu.CompilerParams(use_tc_tiling_on_sc=False,
                                         needs_layout_passes=False),
    scratch_types=[
        pltpu.HBM((2 * _NP, _DH), _bf16),
        pltpu.HBM((2 * _NP, _DH), _bf16),
        pltpu.HBM((2 * _NP, _DH), _bf16),
        pltpu.HBM((2 * _NP, _DH), _bf16),
        pltpu.VMEM_SHARED((_NP, _DH), _bf16),
        pltpu.VMEM((_CH, _C), _i32),
        pltpu.VMEM((_CH, _C), _i32),
        pltpu.VMEM((_CH, _C), _f32),
        pltpu.VMEM((_C, _DH), _bf16),
        pltpu.VMEM((_C, _DH), _bf16),
        pltpu.VMEM((_C, _DH), _bf16),
        pltpu.VMEM((_C, _DH), _bf16),
        pltpu.VMEM((_ZC, _DH), _bf16),
        pltpu.VMEM((_ZC, _DH), _bf16),
        pltpu.VMEM((_ZC, _DH), _bf16),
        pltpu.SemaphoreType.DMA,
        pltpu.SemaphoreType.DMA,
        pltpu.SemaphoreType.DMA,
        pltpu.SemaphoreType.DMA,
        pltpu.SemaphoreType.DMA,
        pltpu.SemaphoreType.DMA,
        pltpu.SemaphoreType.DMA,
        pltpu.SemaphoreType.DMA,
    ],
)(_body)


def kernel(user_embeddings, item_embeddings, edge_index, edge_values):
    row = edge_index[0].astype(_i32).reshape(_E // _C, _C)
    col = edge_index[1].astype(_i32).reshape(_E // _C, _C)
    val2 = edge_values.reshape(_E // _C, _C)
    def stack(t):
        s = jnp.zeros((2 * _NP, _DH), _bf16)
        t = t.astype(_bf16)
        return s.at[:_N].set(t[:, :_DH]).at[_NP:_NP + _N].set(t[:, _DH:])
    out_u, out_i = _gcn(row, col, val2,
                        stack(user_embeddings), stack(item_embeddings))
    def unstack(o):
        o = o.astype(_f32)
        return jnp.concatenate([o[:_N], o[_NP:_NP + _N]], axis=1)
    return (unstack(out_u), unstack(out_i))
